# Initial kernel scaffold; baseline (speedup 1.0000x reference)
import jax
import jax.numpy as jnp
from jax.experimental import pallas as pl


def kernel(x, edge_index, W1l, b1, W1r, W2l, b2, W2r, W3l, b3, W3r):
    raise NotImplementedError("write your pallas kernel here")


# speedup vs baseline: 5.3649x; 5.3649x over previous
import functools

import jax
import jax.numpy as jnp
from jax import lax
from jax.experimental import pallas as pl
from jax.experimental.pallas import tpu as pltpu
from jax.experimental.pallas import tpu_sc as plsc

NC = 2
NS = 16
LANES = 16


@functools.partial(jax.jit, static_argnames=("npad", "d", "e", "with_cnt"))
def _sc_aggregate(x_pad, src, dst, *, npad, d, e, with_cnt):
    nw = NC * NS
    ept = e // nw
    K = 80
    iters = ept // K
    rpt = npad // NS

    mesh = plsc.VectorSubcoreMesh(
        core_axis_name="c", subcore_axis_name="s",
        num_cores=NC, num_subcores=NS)

    out_type = [jax.ShapeDtypeStruct((NC, npad, d), jnp.float32)]
    scratch = [
        pltpu.VMEM_SHARED((npad, d), jnp.float32),
        pltpu.VMEM((K,), jnp.int32),
        pltpu.VMEM((K,), jnp.int32),
        pltpu.VMEM((K, d), jnp.float32),
        pltpu.VMEM((16, d), jnp.float32),
        pltpu.SemaphoreType.DMA,
    ]
    if with_cnt:
        out_type.append(jax.ShapeDtypeStruct((NC, npad), jnp.float32))
        scratch += [
            pltpu.VMEM_SHARED((npad,), jnp.float32),
            pltpu.VMEM((K,), jnp.float32),
            pltpu.VMEM((rpt,), jnp.float32),
        ]

    def body(x_hbm, src_hbm, dst_hbm, out_hbm, *rest):
        if with_cnt:
            cnt_hbm, acc, src_v, dst_v, rows_v, zbuf, sem, hist, ones_v, zstrip = rest
        else:
            acc, src_v, dst_v, rows_v, zbuf, sem = rest
        c = lax.axis_index("c")
        s = lax.axis_index("s")
        wid = c * NS + s
        row0 = s * rpt
        ebase = wid * ept

        def fill_zb(i, _):
            zbuf[i // (d // LANES), pl.ds((i % (d // LANES)) * LANES, LANES)] = (
                jnp.zeros((LANES,), jnp.float32))
            return 0
        lax.fori_loop(0, 16 * (d // LANES), fill_zb, 0)

        def zcp(j, _):
            pltpu.sync_copy(zbuf, acc.at[pl.ds(row0 + j * 16, 16)])
            return 0
        lax.fori_loop(0, rpt // 16, zcp, 0)

        if with_cnt:
            def fill_zs(i, _):
                zstrip[pl.ds(i * LANES, LANES)] = jnp.zeros((LANES,), jnp.float32)
                return 0
            lax.fori_loop(0, rpt // LANES, fill_zs, 0)
            pltpu.sync_copy(zstrip, hist.at[pl.ds(row0, rpt)])

            def fill_ones(i, _):
                ones_v[pl.ds(i * LANES, LANES)] = jnp.ones((LANES,), jnp.float32)
                return 0
            lax.fori_loop(0, K // LANES, fill_ones, 0)

        plsc.subcore_barrier()

        def step(i, _):
            base = ebase + i * K
            pltpu.sync_copy(src_hbm.at[pl.ds(base, K)], src_v)
            pltpu.sync_copy(dst_hbm.at[pl.ds(base, K)], dst_v)
            pltpu.async_copy(x_hbm.at[src_v], rows_v, sem).wait()
            pltpu.sync_copy(rows_v, acc.at[dst_v], add=True)
            if with_cnt:
                pltpu.sync_copy(ones_v, hist.at[dst_v], add=True)
            return 0
        lax.fori_loop(0, iters, step, 0)

        plsc.subcore_barrier()

        pltpu.sync_copy(acc.at[pl.ds(row0, rpt)], out_hbm.at[c, pl.ds(row0, rpt)])
        if with_cnt:
            pltpu.sync_copy(hist.at[pl.ds(row0, rpt)],
                            cnt_hbm.at[c, pl.ds(row0, rpt)])

    return pl.kernel(body, out_type=tuple(out_type), mesh=mesh,
                     scratch_types=tuple(scratch))(x_pad, src, dst)


def _dense_body(s_ref, cnt_ref, x_ref, wl_ref, wr_ref, b_ref, o_ref):
    ssum = s_ref[0] + s_ref[1]
    cnt = cnt_ref[0] + cnt_ref[1]
    inv = 1.0 / jnp.maximum(cnt, 1.0)
    mean = ssum * inv[:, None]
    h = (jnp.dot(mean, wl_ref[...], preferred_element_type=jnp.float32)
         + jnp.dot(x_ref[...], wr_ref[...], preferred_element_type=jnp.float32)
         + b_ref[...])
    o_ref[...] = jnp.maximum(h, 0.0)


@functools.partial(jax.jit, static_argnames=("npad", "d", "bn"))
def _tc_dense(summed, cnt, x_pad, wl, b, wr, *, npad, d, bn):
    grid = (npad // bn,)
    return pl.pallas_call(
        _dense_body,
        grid=grid,
        in_specs=[
            pl.BlockSpec((NC, bn, d), lambda k: (0, k, 0)),
            pl.BlockSpec((NC, bn), lambda k: (0, k)),
            pl.BlockSpec((bn, d), lambda k: (k, 0)),
            pl.BlockSpec((d, d), lambda k: (0, 0)),
            pl.BlockSpec((d, d), lambda k: (0, 0)),
            pl.BlockSpec((1, d), lambda k: (0, 0)),
        ],
        out_specs=pl.BlockSpec((bn, d), lambda k: (k, 0)),
        out_shape=jax.ShapeDtypeStruct((npad, d), jnp.float32),
    )(summed, cnt, x_pad, wl, wr, b.reshape(1, d))


def kernel(x, edge_index, W1l, b1, W1r, W2l, b2, W2r, W3l, b3, W3r):
    n, d = x.shape
    e = edge_index.shape[1]
    npad = ((n + 2047) // 2048) * 2048
    bn = 2048
    src = edge_index[0]
    dst = edge_index[1]
    x_pad = jnp.zeros((npad, d), jnp.float32).at[:n].set(x)

    summed, cnt = _sc_aggregate(x_pad, src, dst, npad=npad, d=d, e=e,
                                with_cnt=True)
    h = _tc_dense(summed, cnt, x_pad, W1l, b1, W1r, npad=npad, d=d, bn=bn)
    (summed,) = _sc_aggregate(h, src, dst, npad=npad, d=d, e=e, with_cnt=False)
    h = _tc_dense(summed, cnt, h, W2l, b2, W2r, npad=npad, d=d, bn=bn)
    (summed,) = _sc_aggregate(h, src, dst, npad=npad, d=d, e=e, with_cnt=False)
    h = _tc_dense(summed, cnt, h, W3l, b3, W3r, npad=npad, d=d, bn=bn)
    return h[:n]
